# baseline (speedup 1.0000x reference)
import functools

import jax
import jax.numpy as jnp
from jax import lax
from jax.experimental import pallas as pl
from jax.experimental.pallas import tpu as pltpu
from jax.experimental.pallas import tpu_sc as plsc

_R, _D = 1_000_000, 64
_N = 16384
_FLAT = _N * _D

_NC, _NS, _L = 2, 16, 16
_NW = _NC * _NS

_CH = 128
_ROWS = _FLAT // _CH

_G = 8
_JG = _D // _G
_SROWS = _ROWS // _G
_SNR = _SROWS // _NW

_K = 256
_BW = _K * _CH
_NB = -(-7813 // _K)
_OB = _JG * _BW
_TSEG = _NB * _OB


def _relayout_body(in_ref, out_ref):
    x = in_ref[...].reshape(_JG, _K, _CH)
    out_ref[...] = jnp.swapaxes(x, 0, 1).reshape(_OB)


@functools.cache
def _relayout_kernel(g):
    return pl.pallas_call(
        _relayout_body,
        grid=(_NB,),
        in_specs=[pl.BlockSpec((_JG, _BW), lambda c: (g, c))],
        out_specs=pl.BlockSpec((_OB,), lambda c: (c,)),
        out_shape=jax.ShapeDtypeStruct((_TSEG,), jnp.float32),
    )


def _make_gather_body(g):
    def _gather_body(t_hbm, c_hbm, out_hbm, ibuf, gbuf, sem):
        wid = lax.axis_index("s") * _NC + lax.axis_index("c")
        grow0 = g * _SROWS + wid * _SNR
        row0 = wid * _SNR
        pltpu.sync_copy(c_hbm.at[pl.ds(grow0, _SNR), :], ibuf)

        def fire(r, carry):
            jconst = ((grow0 + r) >> 7 & (_JG - 1)) * _CH
            for m in range(_CH // _L):
                sl = pl.ds(m * _L, _L)
                v = ibuf[r, sl]
                ibuf[r, sl] = ((v & ~jnp.int32(127)) << 3) + ((v & 127) + jconst)
            pltpu.async_copy(t_hbm.at[ibuf.at[r]], gbuf.at[r], sem)
            return carry

        lax.fori_loop(0, _SNR, fire, 0)
        pltpu.make_async_copy(out_hbm.at[pl.ds(row0, _SNR), :], gbuf, sem).wait()
        pltpu.sync_copy(gbuf, out_hbm.at[pl.ds(row0, _SNR), :])

    return _gather_body


@functools.cache
def _gather_kernel(g):
    mesh = plsc.VectorSubcoreMesh(
        core_axis_name="c", subcore_axis_name="s", num_cores=_NC, num_subcores=_NS
    )
    return pl.kernel(
        _make_gather_body(g),
        mesh=mesh,
        out_type=jax.ShapeDtypeStruct((_SROWS, _CH), jnp.float32),
        scratch_types=[
            pltpu.VMEM((_SNR, _CH), jnp.int32),
            pltpu.VMEM((_SNR, _CH), jnp.float32),
            pltpu.SemaphoreType.DMA,
        ],
    )


def kernel(t, d, c):
    idx = c + jnp.asarray(d, dtype=c.dtype)
    cflat = idx.T.reshape(_ROWS, _CH)
    tt = t.T
    outs = []
    for g in range(_G):
        tseg = _relayout_kernel(g)(tt)
        outs.append(_gather_kernel(g)(tseg, cflat))
    out = jnp.concatenate(outs, axis=0)
    return out.reshape(_D, _N).T


# speedup vs baseline: 1.4093x; 1.4093x over previous
import functools

import jax
import jax.numpy as jnp
from jax import lax
from jax.experimental import pallas as pl
from jax.experimental.pallas import tpu as pltpu
from jax.experimental.pallas import tpu_sc as plsc

_R, _D = 1_000_000, 64
_N = 16384
_FLAT = _N * _D

_NC, _NS, _L = 2, 16, 16
_NW = _NC * _NS

_CH = 128
_ROWS = _FLAT // _CH
_NR = _ROWS // _NW

_K = 256
_BW = _K * _CH
_NB = -(-7813 // _K)
_OB = _D * _BW


def _relayout_body(in_ref, out_ref):
    x = in_ref[...].reshape(_D, _K, _CH)
    out_ref[...] = jnp.swapaxes(x, 0, 1).reshape(_OB)


@functools.cache
def _relayout_kernel():
    return pl.pallas_call(
        _relayout_body,
        grid=(_NB,),
        in_specs=[pl.BlockSpec((_D, _BW), lambda c: (0, c))],
        out_specs=pl.BlockSpec((_OB,), lambda c: (c,)),
        out_shape=jax.ShapeDtypeStruct((_NB * _OB,), jnp.float32),
    )


def _gather_body(t_hbm, c_hbm, out_hbm, ibuf, gbuf, sem):
    wid = lax.axis_index("s") * _NC + lax.axis_index("c")
    row0 = wid * _NR
    pltpu.sync_copy(c_hbm.at[pl.ds(row0, _NR), :], ibuf)

    def fire(r, carry):
        pltpu.async_copy(t_hbm.at[ibuf.at[r]], gbuf.at[r], sem)
        return carry

    lax.fori_loop(0, _NR, fire, 0)
    pltpu.make_async_copy(out_hbm.at[pl.ds(row0, _NR), :], gbuf, sem).wait()
    pltpu.sync_copy(gbuf, out_hbm.at[pl.ds(row0, _NR), :])


@functools.cache
def _gather_kernel():
    mesh = plsc.VectorSubcoreMesh(
        core_axis_name="c", subcore_axis_name="s", num_cores=_NC, num_subcores=_NS
    )
    return pl.kernel(
        _gather_body,
        mesh=mesh,
        out_type=jax.ShapeDtypeStruct((_ROWS, _CH), jnp.float32),
        scratch_types=[
            pltpu.VMEM((_NR, _CH), jnp.int32),
            pltpu.VMEM((_NR, _CH), jnp.float32),
            pltpu.SemaphoreType.DMA,
        ],
    )


def kernel(t, d, c):
    v = (c + jnp.asarray(d, dtype=c.dtype)).T
    jcol = jnp.arange(_D, dtype=jnp.int32)[:, None] * jnp.int32(128)
    phys = ((v & ~jnp.int32(127)) << 6) + ((v & 127) + jcol)
    cflat = phys.reshape(_ROWS, _CH)
    tflat = _relayout_kernel()(t.T)
    out = _gather_kernel()(tflat, cflat)
    return out.reshape(_D, _N).T


# speedup vs baseline: 1.4100x; 1.0005x over previous
import functools

import jax
import jax.numpy as jnp
from jax import lax
from jax.experimental import pallas as pl
from jax.experimental.pallas import tpu as pltpu
from jax.experimental.pallas import tpu_sc as plsc

_R, _D = 1_000_000, 64
_N = 16384
_FLAT = _N * _D

_NC, _NS, _L = 2, 16, 16
_NW = _NC * _NS

_CH = 128
_ROWS = _FLAT // _CH
_NR = _ROWS // _NW

_K = 384
_BW = _K * _CH
_NB = -(-7813 // _K)
_OB = _D * _BW


def _relayout_body(in_ref, out_ref):
    x = in_ref[...].reshape(_D, _K, _CH)
    out_ref[...] = jnp.swapaxes(x, 0, 1).reshape(_OB)


@functools.cache
def _relayout_kernel():
    return pl.pallas_call(
        _relayout_body,
        grid=(_NB,),
        in_specs=[pl.BlockSpec((_D, _BW), lambda c: (0, c))],
        out_specs=pl.BlockSpec((_OB,), lambda c: (c,)),
        out_shape=jax.ShapeDtypeStruct((_NB * _OB,), jnp.float32),
    )


def _gather_body(t_hbm, c_hbm, out_hbm, ibuf, gbuf, sem):
    wid = lax.axis_index("s") * _NC + lax.axis_index("c")
    row0 = wid * _NR
    pltpu.sync_copy(c_hbm.at[pl.ds(row0, _NR), :], ibuf)

    def fire(r, carry):
        jconst = ((row0 + r) >> 7) * _CH
        for m in range(_CH // _L):
            sl = pl.ds(m * _L, _L)
            v = ibuf[r, sl]
            ibuf[r, sl] = ((v & ~jnp.int32(127)) << 6) + ((v & 127) + jconst)
        pltpu.async_copy(t_hbm.at[ibuf.at[r]], gbuf.at[r], sem)
        return carry

    lax.fori_loop(0, _NR, fire, 0)
    pltpu.make_async_copy(out_hbm.at[pl.ds(row0, _NR), :], gbuf, sem).wait()
    pltpu.sync_copy(gbuf, out_hbm.at[pl.ds(row0, _NR), :])


@functools.cache
def _gather_kernel():
    mesh = plsc.VectorSubcoreMesh(
        core_axis_name="c", subcore_axis_name="s", num_cores=_NC, num_subcores=_NS
    )
    return pl.kernel(
        _gather_body,
        mesh=mesh,
        out_type=jax.ShapeDtypeStruct((_ROWS, _CH), jnp.float32),
        scratch_types=[
            pltpu.VMEM((_NR, _CH), jnp.int32),
            pltpu.VMEM((_NR, _CH), jnp.float32),
            pltpu.SemaphoreType.DMA,
        ],
    )


def kernel(t, d, c):
    idx = c + jnp.asarray(d, dtype=c.dtype)
    cflat = idx.T.reshape(_ROWS, _CH)
    tflat = _relayout_kernel()(t.T)
    out = _gather_kernel()(tflat, cflat)
    return out.reshape(_D, _N).T


# speedup vs baseline: 1.4329x; 1.0163x over previous
import functools

import jax
import jax.numpy as jnp
from jax import lax
from jax.experimental import pallas as pl
from jax.experimental.pallas import tpu as pltpu
from jax.experimental.pallas import tpu_sc as plsc

_R, _D = 1_000_000, 64
_N = 16384
_FLAT = _N * _D

_NC, _NS, _L = 2, 16, 16
_NW = _NC * _NS

_CH = 128
_ROWS = _FLAT // _CH

_G = 2
_JG = _D // _G
_SROWS = _ROWS // _G
_SNR = _SROWS // _NW

_K = 384
_BW = _K * _CH
_NB = -(-7813 // _K)
_OB = _JG * _BW
_TSEG = _NB * _OB


def _relayout_body(in_ref, out_ref):
    x = in_ref[...].reshape(_JG, _K, _CH)
    out_ref[...] = jnp.swapaxes(x, 0, 1).reshape(_OB)


@functools.cache
def _relayout_kernel(g):
    return pl.pallas_call(
        _relayout_body,
        grid=(_NB,),
        in_specs=[pl.BlockSpec((_JG, _BW), lambda c: (g, c))],
        out_specs=pl.BlockSpec((_OB,), lambda c: (c,)),
        out_shape=jax.ShapeDtypeStruct((_TSEG,), jnp.float32),
    )


def _make_gather_body(g):
    def _gather_body(t_hbm, c_hbm, out_hbm, ibuf, gbuf, sem):
        wid = lax.axis_index("s") * _NC + lax.axis_index("c")
        grow0 = g * _SROWS + wid * _SNR
        row0 = wid * _SNR
        pltpu.sync_copy(c_hbm.at[pl.ds(grow0, _SNR), :], ibuf)

        def fire(r, carry):
            jconst = (((grow0 + r) >> 7) & (_JG - 1)) * _CH
            for m in range(_CH // _L):
                sl = pl.ds(m * _L, _L)
                v = ibuf[r, sl]
                ibuf[r, sl] = ((v & ~jnp.int32(127)) << 5) + ((v & 127) + jconst)
            pltpu.async_copy(t_hbm.at[ibuf.at[r]], gbuf.at[r], sem)
            return carry

        lax.fori_loop(0, _SNR, fire, 0)
        pltpu.make_async_copy(out_hbm.at[pl.ds(row0, _SNR), :], gbuf, sem).wait()
        pltpu.sync_copy(gbuf, out_hbm.at[pl.ds(row0, _SNR), :])

    return _gather_body


@functools.cache
def _gather_kernel(g):
    mesh = plsc.VectorSubcoreMesh(
        core_axis_name="c", subcore_axis_name="s", num_cores=_NC, num_subcores=_NS
    )
    return pl.kernel(
        _make_gather_body(g),
        mesh=mesh,
        out_type=jax.ShapeDtypeStruct((_SROWS, _CH), jnp.float32),
        scratch_types=[
            pltpu.VMEM((_SNR, _CH), jnp.int32),
            pltpu.VMEM((_SNR, _CH), jnp.float32),
            pltpu.SemaphoreType.DMA,
        ],
    )


def kernel(t, d, c):
    idx = c + jnp.asarray(d, dtype=c.dtype)
    cflat = idx.T.reshape(_ROWS, _CH)
    tt = t.T
    outs = [_gather_kernel(g)(_relayout_kernel(g)(tt), cflat) for g in range(_G)]
    out = jnp.concatenate(outs, axis=0)
    return out.reshape(_D, _N).T
